# baseline (speedup 1.0000x reference)
import jax
import jax.numpy as jnp
from jax.experimental import pallas as pl
from jax.experimental.pallas import tpu as pltpu

_B = 8
_HBLK = 32


def _pos_kernel(col_ref, row_ref, out_ref, scratch, sem):
    s = pl.program_id(0)
    h = pl.program_id(1)
    n_h = pl.num_programs(1)
    step = s * n_h + h
    d, hblk, w = scratch.shape[1], scratch.shape[2], scratch.shape[3]
    parity = step % 2

    def copies(par, hh, ss):
        return [
            pltpu.make_async_copy(
                scratch.at[par],
                out_ref.at[b, pl.ds(ss * d, d), pl.ds(hh * hblk, hblk), :],
                sem.at[par],
            )
            for b in range(_B)
        ]

    @pl.when(step >= 2)
    def _wait_prev():
        for c in copies(parity, h, s):
            c.wait()

    @pl.when(s == 0)
    def _col():
        colT = col_ref[...].T
        scratch[parity] = jnp.broadcast_to(colT[:, None, :], (d, hblk, w))

    @pl.when(s == 1)
    def _row():
        rowT = row_ref[...].T
        scratch[parity] = jnp.broadcast_to(rowT[:, :, None], (d, hblk, w))

    for c in copies(parity, h, s):
        c.start()

    last = 2 * n_h - 1

    @pl.when(step == last)
    def _drain():
        for c in copies(parity, h, s):
            c.wait()

    @pl.when(step == last)
    def _drain_other():
        for c in copies(1 - parity, h, s):
            c.wait()


def kernel(x, row_embed, col_embed):
    B, C, H, W = x.shape
    d = col_embed.shape[1]

    col = col_embed[:W]
    row = row_embed[:H]

    grid = (2, H // _HBLK)
    out = pl.pallas_call(
        _pos_kernel,
        grid=grid,
        in_specs=[
            pl.BlockSpec((W, d), lambda s, h: (0, 0)),
            pl.BlockSpec((_HBLK, d), lambda s, h: (h, 0)),
        ],
        out_specs=pl.BlockSpec(memory_space=pltpu.MemorySpace.HBM),
        out_shape=jax.ShapeDtypeStruct((B, C, H, W), x.dtype),
        scratch_shapes=[
            pltpu.VMEM((2, d, _HBLK, W), jnp.float32),
            pltpu.SemaphoreType.DMA((2,)),
        ],
    )(col, row)
    return out


# speedup vs baseline: 1.0035x; 1.0035x over previous
import jax
import jax.numpy as jnp
from jax.experimental import pallas as pl
from jax.experimental.pallas import tpu as pltpu


def _pos_kernel(col_ref, row_ref, out_ref):
    s = pl.program_id(0)
    d, hblk, w = out_ref.shape[1], out_ref.shape[2], out_ref.shape[3]

    @pl.when(s == 0)
    def _col():
        colT = col_ref[...].T
        out_ref[0] = jnp.broadcast_to(colT[:, None, :], (d, hblk, w))

    @pl.when(s == 1)
    def _row():
        rowT = row_ref[...].T
        out_ref[0] = jnp.broadcast_to(rowT[:, :, None], (d, hblk, w))


def kernel(x, row_embed, col_embed):
    B, C, H, W = x.shape
    d = col_embed.shape[1]
    HBLK = 32

    col = col_embed[:W]
    row = row_embed[:H]

    grid = (2, B, H // HBLK)
    out = pl.pallas_call(
        _pos_kernel,
        grid=grid,
        in_specs=[
            pl.BlockSpec((W, d), lambda s, b, h: (0, 0)),
            pl.BlockSpec((HBLK, d), lambda s, b, h: (h, 0)),
        ],
        out_specs=pl.BlockSpec((1, d, HBLK, W), lambda s, b, h: (b, s, h, 0)),
        out_shape=jax.ShapeDtypeStruct((B, C, H, W), x.dtype),
        compiler_params=pltpu.CompilerParams(
            dimension_semantics=("parallel", "parallel", "arbitrary"),
        ),
    )(col, row)
    return out


# speedup vs baseline: 1.0077x; 1.0042x over previous
import jax
import jax.numpy as jnp
from jax.experimental import pallas as pl
from jax.experimental.pallas import tpu as pltpu

_B = 8


def _pos_kernel(col_ref, row_ref, out_ref, scratch, sem):
    s = pl.program_id(0)
    d, hh, w = scratch.shape[1], scratch.shape[2], scratch.shape[3]

    @pl.when(s == 0)
    def _col():
        colT = col_ref[...].T
        scratch[0] = jnp.broadcast_to(colT[:, None, :], (d, hh, w))

    @pl.when(s == 1)
    def _row():
        rowT = row_ref[...].T
        scratch[1] = jnp.broadcast_to(rowT[:, :, None], (d, hh, w))

    def copies(ss):
        return [
            pltpu.make_async_copy(
                scratch.at[ss],
                out_ref.at[b, pl.ds(ss * d, d), :, :],
                sem.at[ss],
            )
            for b in range(_B)
        ]

    for c in copies(s):
        c.start()

    @pl.when(s == 1)
    def _drain():
        for c in copies(1):
            c.wait()
        for c in copies(0):
            c.wait()


def kernel(x, row_embed, col_embed):
    B, C, H, W = x.shape
    d = col_embed.shape[1]

    col = col_embed[:W]
    row = row_embed[:H]

    out = pl.pallas_call(
        _pos_kernel,
        grid=(2,),
        in_specs=[
            pl.BlockSpec((W, d), lambda s: (0, 0)),
            pl.BlockSpec((H, d), lambda s: (0, 0)),
        ],
        out_specs=pl.BlockSpec(memory_space=pltpu.MemorySpace.HBM),
        out_shape=jax.ShapeDtypeStruct((B, C, H, W), x.dtype),
        scratch_shapes=[
            pltpu.VMEM((2, d, H, W), jnp.float32),
            pltpu.SemaphoreType.DMA((2,)),
        ],
    )(col, row)
    return out
